# baseline (speedup 1.0000x reference)
import functools

import jax
import jax.numpy as jnp
from jax import lax
from jax.experimental import pallas as pl
from jax.experimental.pallas import tpu as pltpu
from jax.experimental.pallas import tpu_sc as plsc

_LANES = 16
_CH = 2048


@functools.lru_cache(maxsize=None)
def _make_scatter_kernel(M, D, B, group):
  mesh = plsc.VectorSubcoreMesh(core_axis_name="c", subcore_axis_name="s")
  nc, ns = mesh.num_cores, mesh.num_subcores
  nw = nc * ns
  cols_per_w = D // nw
  n_ch = B // _CH
  n_groups = _CH // _LANES // group

  @functools.partial(
      pl.kernel,
      out_type=(),
      mesh=mesh,
      scratch_types=[
          pltpu.VMEM((M,), jnp.float32),
          pltpu.VMEM((B,), jnp.int32),
          pltpu.VMEM((_CH,), jnp.float32),
          pltpu.VMEM((_CH,), jnp.float32),
          pltpu.VMEM((_CH,), jnp.int32),
          pltpu.VMEM((_CH,), jnp.int32),
          pltpu.VMEM((_CH,), jnp.float32),
          pltpu.VMEM((_CH,), jnp.float32),
          pltpu.SemaphoreType.DMA,
          pltpu.SemaphoreType.DMA,
          pltpu.SemaphoreType.DMA,
          pltpu.SemaphoreType.DMA,
          pltpu.SemaphoreType.DMA,
      ],
      compiler_params=pltpu.CompilerParams(needs_layout_passes=False),
  )
  def scatter_kernel(idxT, srcT, out, winner, idxcol, srcb0, srcb1,
                     ab0, ab1, vb0, vb1, sio, ssrc0, ssrc1, ssc0, ssc1):
    wid = lax.axis_index("s") * nc + lax.axis_index("c")
    srcbs = (srcb0, srcb1)
    ssrcs = (ssrc0, ssrc1)
    abufs = (ab0, ab1)
    vbufs = (vb0, vb1)
    sscs = (ssc0, ssc1)

    def src_desc(c, ch):
      return pltpu.make_async_copy(
          srcT.at[wid * cols_per_w + c, pl.ds(ch * _CH, _CH)],
          srcbs[ch % 2], ssrcs[ch % 2])

    def sc_desc(gs):
      b = gs % 2
      return pltpu.make_async_copy(vbufs[b], out.at[abufs[b]], sscs[b])

    for c in range(cols_per_w):
      j = wid * cols_per_w + c
      dio = pltpu.make_async_copy(idxT.at[j], idxcol, sio)
      dio.start()
      src_desc(c, 0).start()
      dio.wait()

      for ch in range(n_ch):
        if ch + 1 < n_ch:
          src_desc(c, ch + 1).start()
        src_desc(c, ch).wait()
        srcb = srcbs[ch % 2]

        def p1_body(t, carry, *, _ch=ch, _srcb=srcb):
          base = t * (group * _LANES)
          ent = []
          for k in range(group):
            off = base + k * _LANES
            idxv = idxcol[pl.ds(_ch * _CH + off, _LANES)]
            srcv = _srcb[pl.ds(off, _LANES)]
            _, keep = plsc.scan_count(idxv)
            ent.append((idxv, srcv, keep))
          for a, s, m in ent:
            plsc.store_scatter(winner, [a], s, mask=m)
          return carry

        lax.fori_loop(0, n_groups, p1_body, 0)

      jv = jnp.int32(D) * 0 + (wid * cols_per_w + c) * 1
      for ch in range(n_ch):
        gs = c * n_ch + ch
        b = gs % 2
        if gs >= 2:
          sc_desc(gs - 2).wait()
        abuf, vbuf = abufs[b], vbufs[b]

        def p2_body(t, carry, *, _ch=ch, _abuf=abuf, _vbuf=vbuf):
          base = t * (group * _LANES)
          ent = []
          for k in range(group):
            off = base + k * _LANES
            idxv = idxcol[pl.ds(_ch * _CH + off, _LANES)]
            w = plsc.load_gather(winner, [idxv])
            addr = idxv * D + jv
            ent.append((off, addr, w))
          for off, addr, w in ent:
            _abuf[pl.ds(off, _LANES)] = addr
            _vbuf[pl.ds(off, _LANES)] = w
          return carry

        lax.fori_loop(0, n_groups, p2_body, 0)
        sc_desc(gs).start()

    sc_desc(cols_per_w * n_ch - 2).wait()
    sc_desc(cols_per_w * n_ch - 1).wait()

  return scatter_kernel


def kernel(input, dim, index, src):
  M, D = input.shape
  B = index.shape[0]
  idx = index + jnp.asarray(dim, index.dtype)
  out_ref = jax.new_ref(input.reshape(-1))
  f = _make_scatter_kernel(M, D, B, 8)
  f(idx.T, src.T, out_ref)
  return out_ref[...].reshape(M, D)


# speedup vs baseline: 11.8258x; 11.8258x over previous
import functools

import jax
import jax.numpy as jnp
from jax import lax
from jax.experimental import pallas as pl
from jax.experimental.pallas import tpu as pltpu
from jax.experimental.pallas import tpu_sc as plsc

_LANES = 16
_CH = 4096


@functools.lru_cache(maxsize=None)
def _make_scatter_kernel(M, D, B, group):
  mesh = plsc.VectorSubcoreMesh(core_axis_name="c", subcore_axis_name="s")
  nc, ns = mesh.num_cores, mesh.num_subcores
  nw = nc * ns
  cols_per_w = D // nw
  n_ch = B // _CH
  n_groups = _CH // _LANES // group

  @functools.partial(
      pl.kernel,
      out_type=jax.ShapeDtypeStruct((D, M), jnp.float32),
      mesh=mesh,
      scratch_types=[
          pltpu.VMEM((M,), jnp.float32),
          pltpu.VMEM((_CH,), jnp.int32),
          pltpu.VMEM((_CH,), jnp.int32),
          pltpu.VMEM((_CH,), jnp.float32),
          pltpu.VMEM((_CH,), jnp.float32),
          pltpu.SemaphoreType.DMA,
          pltpu.SemaphoreType.DMA,
          pltpu.SemaphoreType.DMA,
          pltpu.SemaphoreType.DMA,
      ],
      compiler_params=pltpu.CompilerParams(needs_layout_passes=False),
  )
  def scatter_kernel(inpT, idxT, srcT, outT, colbuf, ib0, ib1, sb0, sb1,
                     scl, scs, sio0, sio1):
    wid = lax.axis_index("s") * nc + lax.axis_index("c")
    ibufs = (ib0, ib1)
    sbufs = (sb0, sb1)
    sios = (sio0, sio1)

    def col_j(c):
      return wid * cols_per_w + c

    def load_desc(c):
      return pltpu.make_async_copy(inpT.at[col_j(c)], colbuf, scl)

    def store_desc(c):
      return pltpu.make_async_copy(colbuf, outT.at[col_j(c)], scs)

    def io_descs(c, ch):
      b = ch % 2
      sl = pl.ds(ch * _CH, _CH)
      return (pltpu.make_async_copy(idxT.at[col_j(c), sl], ibufs[b], sios[b]),
              pltpu.make_async_copy(srcT.at[col_j(c), sl], sbufs[b], sios[b]))

    def start_io(c, ch):
      di, dsv = io_descs(c, ch)
      di.start()
      dsv.start()

    def wait_io(c, ch):
      di, dsv = io_descs(c, ch)
      di.wait()
      dsv.wait()

    load_desc(0).start()
    start_io(0, 0)

    for c in range(cols_per_w):
      load_desc(c).wait()
      for ch in range(n_ch):
        if ch + 1 < n_ch:
          start_io(c, ch + 1)
        elif c + 1 < cols_per_w:
          start_io(c + 1, 0)
        wait_io(c, ch)
        ibuf = ibufs[ch % 2]
        sbuf = sbufs[ch % 2]

        def chunk_body(t, carry, *, _ibuf=ibuf, _sbuf=sbuf):
          base = t * (group * _LANES)
          ent = []
          for k in range(group):
            off = base + k * _LANES
            idxv = _ibuf[pl.ds(off, _LANES)]
            srcv = _sbuf[pl.ds(off, _LANES)]
            _, keep = plsc.scan_count(idxv)
            ent.append((idxv, srcv, keep))
          for a, s, m in ent:
            plsc.store_scatter(colbuf, [a], s, mask=m)
          return carry

        lax.fori_loop(0, n_groups, chunk_body, 0)

      store_desc(c).start()
      if c + 1 < cols_per_w:
        store_desc(c).wait()
        load_desc(c + 1).start()

    store_desc(cols_per_w - 1).wait()

  return scatter_kernel


def kernel(input, dim, index, src):
  M, D = input.shape
  B = index.shape[0]
  idx = index + jnp.asarray(dim, index.dtype)
  f = _make_scatter_kernel(M, D, B, 8)
  outT = f(input.T, idx.T, src.T)
  return outT.T
